# Initial kernel scaffold; baseline (speedup 1.0000x reference)
import jax
import jax.numpy as jnp
from jax.experimental import pallas as pl


def kernel(x_email, x_url, x_sender, edge_index_sender_email, edge_index_url_email, edge_index_email_url, W_email, b_email, url_base_w, url_spline_w, sender_base_w, sender_spline_w, sage_se_wl, sage_se_bl, sage_se_wr, sage_ue_wl, sage_ue_bl, sage_ue_wr, sage_eu_wl, sage_eu_bl, sage_eu_wr, bn_gamma, bn_beta, cls_base_w, cls_spline_w):
    raise NotImplementedError("write your pallas kernel here")


# speedup vs baseline: 1.0003x; 1.0003x over previous
import functools

import jax
import jax.numpy as jnp
from jax.experimental import pallas as pl
from jax.experimental.pallas import tpu as pltpu

GRID_SIZE = 5
SPLINE_ORDER = 3
C = GRID_SIZE + SPLINE_ORDER


def _email_embed_body(x_ref, w_ref, b_ref, o_ref):
    x = x_ref[...]
    w = w_ref[...]
    acc = jax.lax.dot_general(x, w, (((1,), (1,)), ((), ())),
                              preferred_element_type=jnp.float32)
    o_ref[...] = jnp.tanh(acc + b_ref[...])


def _email_embed(x, w, b):
    n, k = x.shape
    h = w.shape[0]
    blk = 1000
    return pl.pallas_call(
        _email_embed_body,
        grid=(n // blk,),
        in_specs=[
            pl.BlockSpec((blk, k), lambda i: (i, 0)),
            pl.BlockSpec((h, k), lambda i: (0, 0)),
            pl.BlockSpec((1, h), lambda i: (0, 0)),
        ],
        out_specs=pl.BlockSpec((blk, h), lambda i: (i, 0)),
        out_shape=jax.ShapeDtypeStruct((n, h), jnp.float32),
    )(x, w, b.reshape(1, h))


def _make_grid(in_features):
    h = 2.0 / GRID_SIZE
    g = jnp.arange(-SPLINE_ORDER, GRID_SIZE + SPLINE_ORDER + 1,
                   dtype=jnp.float32) * h - 1.0
    return jnp.broadcast_to(g, (in_features, g.shape[0]))


def _b_splines(x, grid):
    xe = x[..., None]
    bases = ((xe >= grid[:, :-1]) & (xe < grid[:, 1:])).astype(x.dtype)
    for k in range(1, SPLINE_ORDER + 1):
        left = (xe - grid[:, :-(k + 1)]) / (grid[:, k:-1] - grid[:, :-(k + 1)]) * bases[..., :-1]
        right = (grid[:, k + 1:] - xe) / (grid[:, k + 1:] - grid[:, 1:-k]) * bases[..., 1:]
        bases = left + right
    return bases


def _kan_layer(x, base_w, spline_w):
    grid = _make_grid(x.shape[-1])
    base = jax.nn.silu(x) @ base_w.T
    sb = _b_splines(x, grid)
    spline = sb.reshape(x.shape[0], -1) @ spline_w.reshape(spline_w.shape[0], -1).T
    return base + spline


def _sage(x_src, x_dst, edge_index, w_l, b_l, w_r):
    src = edge_index[0]
    dst = edge_index[1]
    msg = x_src[src]
    n_dst = x_dst.shape[0]
    agg = jax.ops.segment_sum(msg, dst, num_segments=n_dst)
    cnt = jax.ops.segment_sum(jnp.ones((dst.shape[0],), msg.dtype), dst,
                              num_segments=n_dst)
    mean = agg / jnp.clip(cnt, 1.0, None)[:, None]
    return mean @ w_l.T + b_l + x_dst @ w_r.T


def kernel(x_email, x_url, x_sender, edge_index_sender_email,
           edge_index_url_email, edge_index_email_url, W_email, b_email,
           url_base_w, url_spline_w, sender_base_w, sender_spline_w,
           sage_se_wl, sage_se_bl, sage_se_wr, sage_ue_wl, sage_ue_bl,
           sage_ue_wr, sage_eu_wl, sage_eu_bl, sage_eu_wr, bn_gamma,
           bn_beta, cls_base_w, cls_spline_w):
    h_email = _email_embed(x_email, W_email, b_email)
    h_url = jnp.tanh(_kan_layer(x_url, url_base_w, url_spline_w))
    h_sender = jnp.tanh(_kan_layer(x_sender, sender_base_w, sender_spline_w))
    out_se = _sage(h_sender, h_email, edge_index_sender_email,
                   sage_se_wl, sage_se_bl, sage_se_wr)
    out_ue = _sage(h_url, h_email, edge_index_url_email,
                   sage_ue_wl, sage_ue_bl, sage_ue_wr)
    gnn_email = (out_se + out_ue) * 0.5
    e = jax.nn.leaky_relu(gnn_email, 0.2)
    combined = jnp.concatenate([e, h_email], axis=-1)
    mu = combined.mean(axis=0)
    var = combined.var(axis=0)
    combined = (combined - mu) / jnp.sqrt(var + 1e-5) * bn_gamma + bn_beta
    return _kan_layer(combined, cls_base_w, cls_spline_w)


# speedup vs baseline: 1.3917x; 1.3913x over previous
import functools

import jax
import jax.numpy as jnp
from jax.experimental import pallas as pl
from jax.experimental.pallas import tpu as pltpu

GRID_SIZE = 5
SPLINE_ORDER = 3
NCOEF = GRID_SIZE + SPLINE_ORDER


def _knots():
    h = 2.0 / GRID_SIZE
    return [(i - SPLINE_ORDER) * h - 1.0
            for i in range(GRID_SIZE + 2 * SPLINE_ORDER + 1)]


def _spline_bases(x):
    g = _knots()
    n = len(g) - 1
    b = [jnp.where((x >= g[j]) & (x < g[j + 1]),
                   jnp.float32(1.0), jnp.float32(0.0)) for j in range(n)]
    for k in range(1, SPLINE_ORDER + 1):
        nb = []
        for j in range(len(b) - 1):
            left = (x - g[j]) * (1.0 / (g[j + k] - g[j])) * b[j]
            right = (g[j + k + 1] - x) * (1.0 / (g[j + k + 1] - g[j + 1])) * b[j + 1]
            nb.append(left + right)
        b = nb
    return b


def _dot(a, b):
    return jax.lax.dot_general(a, b, (((1,), (0,)), ((), ())),
                               preferred_element_type=jnp.float32)


def _email_body(x_ref, w_ref, b_ref, o_ref):
    acc = jax.lax.dot_general(x_ref[...], w_ref[...], (((1,), (1,)), ((), ())),
                              preferred_element_type=jnp.float32)
    o_ref[...] = jnp.tanh(acc + b_ref[...])


def _email_embed(x, w, b):
    n, k = x.shape
    h = w.shape[0]
    blk = 1000
    return pl.pallas_call(
        _email_body,
        grid=(n // blk,),
        in_specs=[
            pl.BlockSpec((blk, k), lambda i: (i, 0)),
            pl.BlockSpec((h, k), lambda i: (0, 0)),
            pl.BlockSpec((1, h), lambda i: (0, 0)),
        ],
        out_specs=pl.BlockSpec((blk, h), lambda i: (i, 0)),
        out_shape=jax.ShapeDtypeStruct((n, h), jnp.float32),
    )(x, w, b.reshape(1, h))


def _url_body(x_ref, bwt_ref, swt_ref, o_ref):
    x = x_ref[...]
    acc = _dot(jax.nn.silu(x), bwt_ref[...])
    b = _spline_bases(x)
    for j in range(NCOEF):
        acc += _dot(b[j], swt_ref[j])
    o_ref[...] = jnp.tanh(acc)


def _url_kan(x, base_w, spline_w):
    n, f = x.shape
    h = base_w.shape[0]
    blk = 2000
    bwt = base_w.T
    swt = jnp.transpose(spline_w, (2, 1, 0))
    return pl.pallas_call(
        _url_body,
        grid=(n // blk,),
        in_specs=[
            pl.BlockSpec((blk, f), lambda i: (i, 0)),
            pl.BlockSpec((f, h), lambda i: (0, 0)),
            pl.BlockSpec((NCOEF, f, h), lambda i: (0, 0, 0)),
        ],
        out_specs=pl.BlockSpec((blk, h), lambda i: (i, 0)),
        out_shape=jax.ShapeDtypeStruct((n, h), jnp.float32),
    )(x, bwt, swt)


def _sender_body(x_ref, bw_ref, sw_ref, o_ref):
    x = x_ref[...]
    acc = jax.nn.silu(x) * bw_ref[...]
    b = _spline_bases(x)
    for j in range(NCOEF):
        acc += b[j] * sw_ref[j]
    o_ref[...] = jnp.tanh(acc)


def _sender_kan(x, base_w, spline_w):
    n, f = x.shape
    h = base_w.shape[0]
    blk = 2000
    bw = base_w.T
    sw = jnp.transpose(spline_w, (2, 1, 0))
    return pl.pallas_call(
        _sender_body,
        grid=(n // blk,),
        in_specs=[
            pl.BlockSpec((blk, f), lambda i: (i, 0)),
            pl.BlockSpec((f, h), lambda i: (0, 0)),
            pl.BlockSpec((NCOEF, f, h), lambda i: (0, 0, 0)),
        ],
        out_specs=pl.BlockSpec((blk, h), lambda i: (i, 0)),
        out_shape=jax.ShapeDtypeStruct((n, h), jnp.float32),
    )(x, bw, sw)


def _combine_body(ase_ref, cse_ref, aue_ref, cue_ref, h_ref,
                  wlse_ref, wlue_ref, wrs_ref, b2_ref, e_ref, st_ref):
    i = pl.program_id(0)
    h = h_ref[...]
    z = _dot(ase_ref[...] / cse_ref[...], wlse_ref[...])
    z += _dot(aue_ref[...] / cue_ref[...], wlue_ref[...])
    z += _dot(h, wrs_ref[...])
    z = z * 0.5 + b2_ref[...]
    e = jnp.where(z >= 0, z, 0.2 * z)
    e_ref[...] = e
    s0 = jnp.concatenate([jnp.sum(e, axis=0), jnp.sum(h, axis=0)])
    s1 = jnp.concatenate([jnp.sum(e * e, axis=0), jnp.sum(h * h, axis=0)])
    blk_stats = jnp.stack([s0, s1])

    @pl.when(i == 0)
    def _():
        st_ref[...] = blk_stats

    @pl.when(i > 0)
    def _():
        st_ref[...] += blk_stats


def _combine_stats(agg_se, cnt_se, agg_ue, cnt_ue, h_email,
                   wl_se, wl_ue, wr_sum, b2):
    n, h = h_email.shape
    blk = 1000
    e, stats = pl.pallas_call(
        _combine_body,
        grid=(n // blk,),
        in_specs=[
            pl.BlockSpec((blk, h), lambda i: (i, 0)),
            pl.BlockSpec((blk, 1), lambda i: (i, 0)),
            pl.BlockSpec((blk, h), lambda i: (i, 0)),
            pl.BlockSpec((blk, 1), lambda i: (i, 0)),
            pl.BlockSpec((blk, h), lambda i: (i, 0)),
            pl.BlockSpec((h, h), lambda i: (0, 0)),
            pl.BlockSpec((h, h), lambda i: (0, 0)),
            pl.BlockSpec((h, h), lambda i: (0, 0)),
            pl.BlockSpec((1, h), lambda i: (0, 0)),
        ],
        out_specs=[
            pl.BlockSpec((blk, h), lambda i: (i, 0)),
            pl.BlockSpec((2, 2 * h), lambda i: (0, 0)),
        ],
        out_shape=[
            jax.ShapeDtypeStruct((n, h), jnp.float32),
            jax.ShapeDtypeStruct((2, 2 * h), jnp.float32),
        ],
    )(agg_se, cnt_se, agg_ue, cnt_ue, h_email, wl_se, wl_ue, wr_sum, b2)
    return e, stats


def _cls_body(e_ref, h_ref, sc_ref, sh_ref, bwt_ref, swt_ref, o_ref):
    c = jnp.concatenate([e_ref[...], h_ref[...]], axis=1)
    c = c * sc_ref[...] + sh_ref[...]
    acc = _dot(jax.nn.silu(c), bwt_ref[...])
    b = _spline_bases(c)
    for j in range(NCOEF):
        acc += _dot(b[j], swt_ref[j])
    o_ref[...] = acc


def _classifier(e, h_email, scale, shift, cls_base_w, cls_spline_w):
    n, h = h_email.shape
    f = 2 * h
    out = cls_base_w.shape[0]
    blk = 400
    bwt = cls_base_w.T
    swt = jnp.transpose(cls_spline_w, (2, 1, 0))
    return pl.pallas_call(
        _cls_body,
        grid=(n // blk,),
        in_specs=[
            pl.BlockSpec((blk, h), lambda i: (i, 0)),
            pl.BlockSpec((blk, h), lambda i: (i, 0)),
            pl.BlockSpec((1, f), lambda i: (0, 0)),
            pl.BlockSpec((1, f), lambda i: (0, 0)),
            pl.BlockSpec((f, out), lambda i: (0, 0)),
            pl.BlockSpec((NCOEF, f, out), lambda i: (0, 0, 0)),
        ],
        out_specs=pl.BlockSpec((blk, out), lambda i: (i, 0)),
        out_shape=jax.ShapeDtypeStruct((n, out), jnp.float32),
    )(e, h_email, scale, shift, bwt, swt)


def _segment_agg(x_src, edge_index, n_dst):
    src = edge_index[0]
    dst = edge_index[1]
    msg = x_src[src]
    agg = jax.ops.segment_sum(msg, dst, num_segments=n_dst)
    cnt = jax.ops.segment_sum(jnp.ones((dst.shape[0],), msg.dtype), dst,
                              num_segments=n_dst)
    return agg, cnt


def kernel(x_email, x_url, x_sender, edge_index_sender_email,
           edge_index_url_email, edge_index_email_url, W_email, b_email,
           url_base_w, url_spline_w, sender_base_w, sender_spline_w,
           sage_se_wl, sage_se_bl, sage_se_wr, sage_ue_wl, sage_ue_bl,
           sage_ue_wr, sage_eu_wl, sage_eu_bl, sage_eu_wr, bn_gamma,
           bn_beta, cls_base_w, cls_spline_w):
    n_email = x_email.shape[0]
    h = W_email.shape[0]

    h_email = _email_embed(x_email, W_email, b_email)
    h_url = _url_kan(x_url, url_base_w, url_spline_w)
    h_sender = _sender_kan(x_sender, sender_base_w, sender_spline_w)

    agg_se, cnt_se = _segment_agg(h_sender, edge_index_sender_email, n_email)
    agg_ue, cnt_ue = _segment_agg(h_url, edge_index_url_email, n_email)

    cse = jnp.clip(cnt_se, 1.0, None).reshape(n_email, 1)
    cue = jnp.clip(cnt_ue, 1.0, None).reshape(n_email, 1)
    wr_sum = (sage_se_wr + sage_ue_wr).T
    b2 = (0.5 * (sage_se_bl + sage_ue_bl)).reshape(1, h)
    e, stats = _combine_stats(agg_se, cse, agg_ue, cue, h_email,
                              sage_se_wl.T, sage_ue_wl.T, wr_sum, b2)

    inv_n = 1.0 / n_email
    mu = stats[0] * inv_n
    var = stats[1] * inv_n - mu * mu
    scale = (bn_gamma / jnp.sqrt(var + 1e-5)).reshape(1, 2 * h)
    shift = (bn_beta - mu * scale[0]).reshape(1, 2 * h)

    return _classifier(e, h_email, scale, shift, cls_base_w, cls_spline_w)


# speedup vs baseline: 1.7442x; 1.2533x over previous
import functools

import jax
import jax.numpy as jnp
from jax.experimental import pallas as pl
from jax.experimental.pallas import tpu as pltpu

GRID_SIZE = 5
SPLINE_ORDER = 3
NCOEF = GRID_SIZE + SPLINE_ORDER


def _knots():
    h = 2.0 / GRID_SIZE
    return [(i - SPLINE_ORDER) * h - 1.0
            for i in range(GRID_SIZE + 2 * SPLINE_ORDER + 1)]


def _spline_bases(x):
    g = _knots()
    n = len(g) - 1
    b = [jnp.where((x >= g[j]) & (x < g[j + 1]),
                   jnp.float32(1.0), jnp.float32(0.0)) for j in range(n)]
    for k in range(1, SPLINE_ORDER + 1):
        nb = []
        for j in range(len(b) - 1):
            left = (x - g[j]) * (1.0 / (g[j + k] - g[j])) * b[j]
            right = (g[j + k + 1] - x) * (1.0 / (g[j + k + 1] - g[j + 1])) * b[j + 1]
            nb.append(left + right)
        b = nb
    return b


def _dot(a, b):
    return jax.lax.dot_general(a, b, (((1,), (0,)), ((), ())),
                               preferred_element_type=jnp.float32)


def _email_body(x_ref, w_ref, b_ref, o_ref):
    acc = jax.lax.dot_general(x_ref[...], w_ref[...], (((1,), (1,)), ((), ())),
                              preferred_element_type=jnp.float32)
    o_ref[...] = jnp.tanh(acc + b_ref[...])


def _email_embed(x, w, b):
    n, k = x.shape
    h = w.shape[0]
    blk = 1000
    return pl.pallas_call(
        _email_body,
        grid=(n // blk,),
        in_specs=[
            pl.BlockSpec((blk, k), lambda i: (i, 0)),
            pl.BlockSpec((h, k), lambda i: (0, 0)),
            pl.BlockSpec((1, h), lambda i: (0, 0)),
        ],
        out_specs=pl.BlockSpec((blk, h), lambda i: (i, 0)),
        out_shape=jax.ShapeDtypeStruct((n, h), jnp.float32),
    )(x, w, b.reshape(1, h))


def _url_body(x_ref, bwt_ref, swt_ref, o_ref):
    x = x_ref[...]
    acc = _dot(jax.nn.silu(x), bwt_ref[...])
    b = _spline_bases(x)
    for j in range(NCOEF):
        acc += _dot(b[j], swt_ref[j])
    o_ref[...] = jnp.tanh(acc)


def _url_kan(x, base_w, spline_w):
    n, f = x.shape
    h = base_w.shape[0]
    blk = 2000
    bwt = base_w.T
    swt = jnp.transpose(spline_w, (2, 1, 0))
    return pl.pallas_call(
        _url_body,
        grid=(n // blk,),
        in_specs=[
            pl.BlockSpec((blk, f), lambda i: (i, 0)),
            pl.BlockSpec((f, h), lambda i: (0, 0)),
            pl.BlockSpec((NCOEF, f, h), lambda i: (0, 0, 0)),
        ],
        out_specs=pl.BlockSpec((blk, h), lambda i: (i, 0)),
        out_shape=jax.ShapeDtypeStruct((n, h), jnp.float32),
    )(x, bwt, swt)


def _sender_body(x_ref, bw_ref, sw_ref, o_ref):
    x = x_ref[...]
    acc = jax.nn.silu(x) * bw_ref[...]
    b = _spline_bases(x)
    for j in range(NCOEF):
        acc += b[j] * sw_ref[j]
    o_ref[...] = jnp.tanh(acc)


def _sender_kan(x, base_w, spline_w):
    n, f = x.shape
    h = base_w.shape[0]
    blk = 2000
    bw = base_w.T
    sw = jnp.transpose(spline_w, (2, 1, 0))
    return pl.pallas_call(
        _sender_body,
        grid=(n // blk,),
        in_specs=[
            pl.BlockSpec((blk, f), lambda i: (i, 0)),
            pl.BlockSpec((f, h), lambda i: (0, 0)),
            pl.BlockSpec((NCOEF, f, h), lambda i: (0, 0, 0)),
        ],
        out_specs=pl.BlockSpec((blk, h), lambda i: (i, 0)),
        out_shape=jax.ShapeDtypeStruct((n, h), jnp.float32),
    )(x, bw, sw)


def _combine_body(ase_ref, cse_ref, aue_ref, cue_ref, h_ref,
                  wlse_ref, wlue_ref, wrs_ref, b2_ref, e_ref, st_ref):
    i = pl.program_id(0)
    h = h_ref[...]
    z = _dot(ase_ref[...] / cse_ref[...], wlse_ref[...])
    z += _dot(aue_ref[...] / cue_ref[...], wlue_ref[...])
    z += _dot(h, wrs_ref[...])
    z = z * 0.5 + b2_ref[...]
    e = jnp.where(z >= 0, z, 0.2 * z)
    e_ref[...] = e
    s0 = jnp.concatenate([jnp.sum(e, axis=0), jnp.sum(h, axis=0)])
    s1 = jnp.concatenate([jnp.sum(e * e, axis=0), jnp.sum(h * h, axis=0)])
    blk_stats = jnp.stack([s0, s1])

    @pl.when(i == 0)
    def _():
        st_ref[...] = blk_stats

    @pl.when(i > 0)
    def _():
        st_ref[...] += blk_stats


def _combine_stats(agg_se, cnt_se, agg_ue, cnt_ue, h_email,
                   wl_se, wl_ue, wr_sum, b2):
    n, h = h_email.shape
    blk = 1000
    e, stats = pl.pallas_call(
        _combine_body,
        grid=(n // blk,),
        in_specs=[
            pl.BlockSpec((blk, h), lambda i: (i, 0)),
            pl.BlockSpec((blk, 1), lambda i: (i, 0)),
            pl.BlockSpec((blk, h), lambda i: (i, 0)),
            pl.BlockSpec((blk, 1), lambda i: (i, 0)),
            pl.BlockSpec((blk, h), lambda i: (i, 0)),
            pl.BlockSpec((h, h), lambda i: (0, 0)),
            pl.BlockSpec((h, h), lambda i: (0, 0)),
            pl.BlockSpec((h, h), lambda i: (0, 0)),
            pl.BlockSpec((1, h), lambda i: (0, 0)),
        ],
        out_specs=[
            pl.BlockSpec((blk, h), lambda i: (i, 0)),
            pl.BlockSpec((2, 2 * h), lambda i: (0, 0)),
        ],
        out_shape=[
            jax.ShapeDtypeStruct((n, h), jnp.float32),
            jax.ShapeDtypeStruct((2, 2 * h), jnp.float32),
        ],
    )(agg_se, cnt_se, agg_ue, cnt_ue, h_email, wl_se, wl_ue, wr_sum, b2)
    return e, stats


def _cls_body(e_ref, h_ref, sc_ref, sh_ref, bwt_ref, swt_ref, o_ref):
    c = jnp.concatenate([e_ref[...], h_ref[...]], axis=1)
    c = c * sc_ref[...] + sh_ref[...]
    acc = _dot(jax.nn.silu(c), bwt_ref[...])
    b = _spline_bases(c)
    for j in range(NCOEF):
        acc += _dot(b[j], swt_ref[j])
    o_ref[...] = acc


def _classifier(e, h_email, scale, shift, cls_base_w, cls_spline_w):
    n, h = h_email.shape
    f = 2 * h
    out = cls_base_w.shape[0]
    blk = 400
    bwt = cls_base_w.T
    swt = jnp.transpose(cls_spline_w, (2, 1, 0))
    return pl.pallas_call(
        _cls_body,
        grid=(n // blk,),
        in_specs=[
            pl.BlockSpec((blk, h), lambda i: (i, 0)),
            pl.BlockSpec((blk, h), lambda i: (i, 0)),
            pl.BlockSpec((1, f), lambda i: (0, 0)),
            pl.BlockSpec((1, f), lambda i: (0, 0)),
            pl.BlockSpec((f, out), lambda i: (0, 0)),
            pl.BlockSpec((NCOEF, f, out), lambda i: (0, 0, 0)),
        ],
        out_specs=pl.BlockSpec((blk, out), lambda i: (i, 0)),
        out_shape=jax.ShapeDtypeStruct((n, out), jnp.float32),
    )(e, h_email, scale, shift, bwt, swt)


_E = 200000
_NSUB = 16
_ESH = _E // _NSUB
_CH = 128
_NCHUNK = 98
_EPAD = _NCHUNK * _CH
_NPAD = 50048
_FB = 16
_FROWS = _NPAD // _NSUB


_PROWS = 8344
_NPADA = 6 * _PROWS
_SLAB = 8352
_FL = 1192
_NFL = _PROWS // _FL
_CHA = 64
_NCHA = _EPAD // _CHA


def _sc_agg_body(tbl_ref, src3_ref, d0_ref, d1_ref, d2_ref, d3_ref,
                 d4_ref, d5_ref, zeros_ref, agg_ref,
                 srcv, dstv, buf, sema, slab):
    from jax import lax
    from jax.experimental.pallas import tpu_sc as plsc

    c = lax.axis_index("c")
    s = lax.axis_index("s")
    pltpu.sync_copy(src3_ref.at[s], srcv)

    def zero_slab():
        for j in range(5):
            cid = s + _NSUB * j
            @pl.when(cid < 65)
            def _():
                pltpu.sync_copy(zeros_ref.at[pl.ds(cid * _CH, _CH)],
                                slab.at[pl.ds(cid * _CH, _CH)])
            @pl.when(cid == 65)
            def _():
                pltpu.sync_copy(zeros_ref.at[pl.ds(65 * _CH, _SLAB - 65 * _CH)],
                                slab.at[pl.ds(65 * _CH, _SLAB - 65 * _CH)])

    def accum(tbl):
        def _c(j, _):
            pltpu.async_copy(tbl.at[srcv.at[j]], buf, sema).wait()
            pltpu.sync_copy(buf, slab.at[dstv.at[j]], add=True)
            return 0
        lax.fori_loop(0, _NCHUNK, _c, 0)

    def flush(out, plo):
        for j in range(5):
            cid = s + _NSUB * j
            @pl.when(cid < _NFL)
            def _():
                off = cid * _FL
                pltpu.sync_copy(slab.at[pl.ds(off, _FL)],
                                out.at[pl.ds(plo + off, _FL)])

    def run_round(tbl, out, dst3, plo):
        zero_slab()
        pltpu.sync_copy(dst3.at[s], dstv)
        plsc.subcore_barrier()
        accum(tbl)
        plsc.subcore_barrier()
        flush(out, plo)
        plsc.subcore_barrier()

    @pl.when(c == 0)
    def _():
        run_round(tbl_ref, agg_ref, d0_ref, 0)
        run_round(tbl_ref, agg_ref, d1_ref, _PROWS)
        run_round(tbl_ref, agg_ref, d2_ref, 2 * _PROWS)

    @pl.when(c == 1)
    def _():
        run_round(tbl_ref, agg_ref, d3_ref, 3 * _PROWS)
        run_round(tbl_ref, agg_ref, d4_ref, 4 * _PROWS)
        run_round(tbl_ref, agg_ref, d5_ref, 5 * _PROWS)


def _pad_edges(ei, n_src):
    npad = _EPAD - _ESH
    pad_s = ((jnp.arange(npad, dtype=jnp.int32) * 97) % n_src)[None, :]
    pad_d = (50000 + (jnp.arange(npad, dtype=jnp.int32) % 48))[None, :]
    src = jnp.concatenate(
        [ei[0].reshape(_NSUB, _ESH),
         jnp.broadcast_to(pad_s, (_NSUB, npad))], axis=1)
    dst = jnp.concatenate(
        [ei[1].reshape(_NSUB, _ESH),
         jnp.broadcast_to(pad_d, (_NSUB, npad))], axis=1)
    return src, dst


def _sc_aggregate(h_sender, h_url, ei_se, ei_ue):
    from jax.experimental.pallas import tpu_sc as plsc

    mesh = plsc.VectorSubcoreMesh(core_axis_name="c", subcore_axis_name="s")
    sse, dse = _pad_edges(ei_se, 10000)
    sue, due = _pad_edges(ei_ue, 50000)

    def locals_for(dst):
        out = []
        for p in range(6):
            plo = p * _PROWS
            inr = (dst >= plo) & (dst < plo + _PROWS)
            out.append(jnp.where(inr, dst - plo,
                                 _PROWS + (dst & 7)).reshape(
                                     _NSUB, _NCHUNK, _CH))
        return out

    zeros = jnp.zeros((_SLAB, 128), jnp.float32)
    agg_kernel = functools.partial(
        pl.kernel, mesh=mesh,
        out_type=jax.ShapeDtypeStruct((_NPADA, 128), jnp.float32),
        scratch_types=[
            pltpu.VMEM((_NCHUNK, _CH), jnp.int32),
            pltpu.VMEM((_NCHUNK, _CH), jnp.int32),
            pltpu.VMEM((_CH, 128), jnp.float32),
            pltpu.SemaphoreType.DMA,
            pltpu.VMEM_SHARED((_SLAB, 128), jnp.float32),
        ])(_sc_agg_body)

    agg_se = agg_kernel(h_sender, sse.reshape(_NSUB, _NCHUNK, _CH),
                        *locals_for(dse), zeros)
    dep = (agg_se[0, 0] * 0.0).astype(jnp.int32)
    agg_ue = agg_kernel(h_url, sue.reshape(_NSUB, _NCHUNK, _CH) + dep,
                        *locals_for(due), zeros)

    ones_e = jnp.ones((_E,), jnp.float32)
    cnt_se = jax.ops.segment_sum(ones_e, ei_se[1], num_segments=50000)
    cnt_ue = jax.ops.segment_sum(ones_e, ei_ue[1], num_segments=50000)
    return agg_se, cnt_se[:, None], agg_ue, cnt_ue[:, None]


def kernel(x_email, x_url, x_sender, edge_index_sender_email,
           edge_index_url_email, edge_index_email_url, W_email, b_email,
           url_base_w, url_spline_w, sender_base_w, sender_spline_w,
           sage_se_wl, sage_se_bl, sage_se_wr, sage_ue_wl, sage_ue_bl,
           sage_ue_wr, sage_eu_wl, sage_eu_bl, sage_eu_wr, bn_gamma,
           bn_beta, cls_base_w, cls_spline_w):
    n_email = x_email.shape[0]
    h = W_email.shape[0]

    h_email = _email_embed(x_email, W_email, b_email)
    h_url = _url_kan(x_url, url_base_w, url_spline_w)
    h_sender = _sender_kan(x_sender, sender_base_w, sender_spline_w)

    agg_se, cnt_se, agg_ue, cnt_ue = _sc_aggregate(
        h_sender, h_url, edge_index_sender_email, edge_index_url_email)

    cse = jnp.clip(cnt_se[:n_email, 0], 1.0, None).reshape(n_email, 1)
    cue = jnp.clip(cnt_ue[:n_email, 0], 1.0, None).reshape(n_email, 1)
    wr_sum = (sage_se_wr + sage_ue_wr).T
    b2 = (0.5 * (sage_se_bl + sage_ue_bl)).reshape(1, h)
    e, stats = _combine_stats(agg_se, cse, agg_ue, cue, h_email,
                              sage_se_wl.T, sage_ue_wl.T, wr_sum, b2)

    inv_n = 1.0 / n_email
    mu = stats[0] * inv_n
    var = stats[1] * inv_n - mu * mu
    scale = (bn_gamma / jnp.sqrt(var + 1e-5)).reshape(1, 2 * h)
    shift = (bn_beta - mu * scale[0]).reshape(1, 2 * h)

    return _classifier(e, h_email, scale, shift, cls_base_w, cls_spline_w)


# speedup vs baseline: 2.0512x; 1.1760x over previous
import functools

import jax
import jax.numpy as jnp
from jax.experimental import pallas as pl
from jax.experimental.pallas import tpu as pltpu

GRID_SIZE = 5
SPLINE_ORDER = 3
NCOEF = GRID_SIZE + SPLINE_ORDER


def _knots():
    h = 2.0 / GRID_SIZE
    return [(i - SPLINE_ORDER) * h - 1.0
            for i in range(GRID_SIZE + 2 * SPLINE_ORDER + 1)]


def _spline_bases(x):
    g = _knots()
    n = len(g) - 1
    b = [jnp.where((x >= g[j]) & (x < g[j + 1]),
                   jnp.float32(1.0), jnp.float32(0.0)) for j in range(n)]
    for k in range(1, SPLINE_ORDER + 1):
        nb = []
        for j in range(len(b) - 1):
            left = (x - g[j]) * (1.0 / (g[j + k] - g[j])) * b[j]
            right = (g[j + k + 1] - x) * (1.0 / (g[j + k + 1] - g[j + 1])) * b[j + 1]
            nb.append(left + right)
        b = nb
    return b


def _dot(a, b):
    return jax.lax.dot_general(a, b, (((1,), (0,)), ((), ())),
                               preferred_element_type=jnp.float32)


def _email_body(x_ref, w_ref, b_ref, o_ref):
    acc = jax.lax.dot_general(x_ref[...], w_ref[...], (((1,), (1,)), ((), ())),
                              preferred_element_type=jnp.float32)
    o_ref[...] = jnp.tanh(acc + b_ref[...])


def _email_embed(x, w, b):
    n, k = x.shape
    h = w.shape[0]
    blk = 1000
    return pl.pallas_call(
        _email_body,
        grid=(n // blk,),
        in_specs=[
            pl.BlockSpec((blk, k), lambda i: (i, 0)),
            pl.BlockSpec((h, k), lambda i: (0, 0)),
            pl.BlockSpec((1, h), lambda i: (0, 0)),
        ],
        out_specs=pl.BlockSpec((blk, h), lambda i: (i, 0)),
        out_shape=jax.ShapeDtypeStruct((n, h), jnp.float32),
    )(x, w, b.reshape(1, h))


def _url_body(x_ref, bwt_ref, swt_ref, o_ref):
    x = x_ref[...]
    acc = _dot(jax.nn.silu(x), bwt_ref[...])
    b = _spline_bases(x)
    for j in range(NCOEF):
        acc += _dot(b[j], swt_ref[j])
    o_ref[...] = jnp.tanh(acc)


def _url_kan(x, base_w, spline_w):
    n, f = x.shape
    h = base_w.shape[0]
    blk = 2000
    bwt = base_w.T
    swt = jnp.transpose(spline_w, (2, 1, 0))
    return pl.pallas_call(
        _url_body,
        grid=(n // blk,),
        in_specs=[
            pl.BlockSpec((blk, f), lambda i: (i, 0)),
            pl.BlockSpec((f, h), lambda i: (0, 0)),
            pl.BlockSpec((NCOEF, f, h), lambda i: (0, 0, 0)),
        ],
        out_specs=pl.BlockSpec((blk, h), lambda i: (i, 0)),
        out_shape=jax.ShapeDtypeStruct((n, h), jnp.float32),
    )(x, bwt, swt)


def _sender_body(x_ref, bw_ref, sw_ref, o_ref):
    x = x_ref[...]
    acc = jax.nn.silu(x) * bw_ref[...]
    b = _spline_bases(x)
    for j in range(NCOEF):
        acc += b[j] * sw_ref[j]
    o_ref[...] = jnp.tanh(acc)


def _sender_kan(x, base_w, spline_w):
    n, f = x.shape
    h = base_w.shape[0]
    blk = 2000
    bw = base_w.T
    sw = jnp.transpose(spline_w, (2, 1, 0))
    return pl.pallas_call(
        _sender_body,
        grid=(n // blk,),
        in_specs=[
            pl.BlockSpec((blk, f), lambda i: (i, 0)),
            pl.BlockSpec((f, h), lambda i: (0, 0)),
            pl.BlockSpec((NCOEF, f, h), lambda i: (0, 0, 0)),
        ],
        out_specs=pl.BlockSpec((blk, h), lambda i: (i, 0)),
        out_shape=jax.ShapeDtypeStruct((n, h), jnp.float32),
    )(x, bw, sw)


def _combine_body(ase_ref, cse_ref, aue_ref, cue_ref, h_ref,
                  wlse_ref, wlue_ref, wrs_ref, b2_ref, e_ref, st_ref):
    i = pl.program_id(0)
    h = h_ref[...]
    z = _dot(ase_ref[...] / cse_ref[...], wlse_ref[...])
    z += _dot(aue_ref[...] / cue_ref[...], wlue_ref[...])
    z += _dot(h, wrs_ref[...])
    z = z * 0.5 + b2_ref[...]
    e = jnp.where(z >= 0, z, 0.2 * z)
    e_ref[...] = e
    s0 = jnp.concatenate([jnp.sum(e, axis=0), jnp.sum(h, axis=0)])
    s1 = jnp.concatenate([jnp.sum(e * e, axis=0), jnp.sum(h * h, axis=0)])
    blk_stats = jnp.stack([s0, s1])

    @pl.when(i == 0)
    def _():
        st_ref[...] = blk_stats

    @pl.when(i > 0)
    def _():
        st_ref[...] += blk_stats


def _combine_stats(agg_se, cnt_se, agg_ue, cnt_ue, h_email,
                   wl_se, wl_ue, wr_sum, b2):
    n, h = h_email.shape
    blk = 1000
    e, stats = pl.pallas_call(
        _combine_body,
        grid=(n // blk,),
        in_specs=[
            pl.BlockSpec((blk, h), lambda i: (i, 0)),
            pl.BlockSpec((blk, 1), lambda i: (i, 0)),
            pl.BlockSpec((blk, h), lambda i: (i, 0)),
            pl.BlockSpec((blk, 1), lambda i: (i, 0)),
            pl.BlockSpec((blk, h), lambda i: (i, 0)),
            pl.BlockSpec((h, h), lambda i: (0, 0)),
            pl.BlockSpec((h, h), lambda i: (0, 0)),
            pl.BlockSpec((h, h), lambda i: (0, 0)),
            pl.BlockSpec((1, h), lambda i: (0, 0)),
        ],
        out_specs=[
            pl.BlockSpec((blk, h), lambda i: (i, 0)),
            pl.BlockSpec((2, 2 * h), lambda i: (0, 0)),
        ],
        out_shape=[
            jax.ShapeDtypeStruct((n, h), jnp.float32),
            jax.ShapeDtypeStruct((2, 2 * h), jnp.float32),
        ],
    )(agg_se, cnt_se, agg_ue, cnt_ue, h_email, wl_se, wl_ue, wr_sum, b2)
    return e, stats


def _cls_body(e_ref, h_ref, sc_ref, sh_ref, bwt_ref, swt_ref, o_ref):
    c = jnp.concatenate([e_ref[...], h_ref[...]], axis=1)
    c = c * sc_ref[...] + sh_ref[...]
    acc = _dot(jax.nn.silu(c), bwt_ref[...])
    b = _spline_bases(c)
    for j in range(NCOEF):
        acc += _dot(b[j], swt_ref[j])
    o_ref[...] = acc


def _classifier(e, h_email, scale, shift, cls_base_w, cls_spline_w):
    n, h = h_email.shape
    f = 2 * h
    out = cls_base_w.shape[0]
    blk = 400
    bwt = cls_base_w.T
    swt = jnp.transpose(cls_spline_w, (2, 1, 0))
    return pl.pallas_call(
        _cls_body,
        grid=(n // blk,),
        in_specs=[
            pl.BlockSpec((blk, h), lambda i: (i, 0)),
            pl.BlockSpec((blk, h), lambda i: (i, 0)),
            pl.BlockSpec((1, f), lambda i: (0, 0)),
            pl.BlockSpec((1, f), lambda i: (0, 0)),
            pl.BlockSpec((f, out), lambda i: (0, 0)),
            pl.BlockSpec((NCOEF, f, out), lambda i: (0, 0, 0)),
        ],
        out_specs=pl.BlockSpec((blk, out), lambda i: (i, 0)),
        out_shape=jax.ShapeDtypeStruct((n, out), jnp.float32),
    )(e, h_email, scale, shift, bwt, swt)


_E = 200000
_NSUB = 16
_ESH = _E // _NSUB
_CH = 128
_NCHUNK = 98
_EPAD = _NCHUNK * _CH
_NPAD = 50048
_FB = 16
_FROWS = _NPAD // _NSUB


_PROWS = 8344
_NPADA = 6 * _PROWS
_SLAB = 8352
_FL = 1192
_NFL = _PROWS // _FL
_CHA = 64
_NCHA = _EPAD // _CHA


def _sc_agg_body(tbl_ref, src3_ref, d0_ref, d1_ref, d2_ref, d3_ref,
                 d4_ref, d5_ref, zeros_ref, agg_ref,
                 srcv, dstv, bufa, bufb, sema, semb, slab):
    from jax import lax
    from jax.experimental.pallas import tpu_sc as plsc

    c = lax.axis_index("c")
    s = lax.axis_index("s")
    pltpu.sync_copy(src3_ref.at[s], srcv)

    def zero_slab():
        for j in range(5):
            cid = s + _NSUB * j
            @pl.when(cid < 65)
            def _():
                pltpu.sync_copy(zeros_ref.at[pl.ds(cid * _CH, _CH)],
                                slab.at[pl.ds(cid * _CH, _CH)])
            @pl.when(cid == 65)
            def _():
                pltpu.sync_copy(zeros_ref.at[pl.ds(65 * _CH, _SLAB - 65 * _CH)],
                                slab.at[pl.ds(65 * _CH, _SLAB - 65 * _CH)])

    def accum(tbl):
        pltpu.async_copy(tbl.at[srcv.at[0]], bufa, sema)
        def _c(jj, _):
            pltpu.make_async_copy(tbl.at[srcv.at[0]], bufa, sema).wait()
            pltpu.async_copy(tbl.at[srcv.at[2 * jj + 1]], bufb, semb)
            pltpu.sync_copy(bufa, slab.at[dstv.at[2 * jj]], add=True)
            pltpu.make_async_copy(tbl.at[srcv.at[0]], bufb, semb).wait()
            @pl.when(jj < _NCHUNK // 2 - 1)
            def _():
                pltpu.async_copy(tbl.at[srcv.at[2 * jj + 2]], bufa, sema)
            pltpu.sync_copy(bufb, slab.at[dstv.at[2 * jj + 1]], add=True)
            return 0
        lax.fori_loop(0, _NCHUNK // 2, _c, 0)

    def flush(out, plo):
        for j in range(5):
            cid = s + _NSUB * j
            @pl.when(cid < _NFL)
            def _():
                off = cid * _FL
                pltpu.sync_copy(slab.at[pl.ds(off, _FL)],
                                out.at[pl.ds(plo + off, _FL)])

    def run_round(tbl, out, dst3, plo):
        zero_slab()
        pltpu.sync_copy(dst3.at[s], dstv)
        plsc.subcore_barrier()
        accum(tbl)
        plsc.subcore_barrier()
        flush(out, plo)
        plsc.subcore_barrier()

    @pl.when(c == 0)
    def _():
        run_round(tbl_ref, agg_ref, d0_ref, 0)
        run_round(tbl_ref, agg_ref, d1_ref, _PROWS)
        run_round(tbl_ref, agg_ref, d2_ref, 2 * _PROWS)

    @pl.when(c == 1)
    def _():
        run_round(tbl_ref, agg_ref, d3_ref, 3 * _PROWS)
        run_round(tbl_ref, agg_ref, d4_ref, 4 * _PROWS)
        run_round(tbl_ref, agg_ref, d5_ref, 5 * _PROWS)


def _pad_edges(ei, n_src):
    npad = _EPAD - _ESH
    pad_s = ((jnp.arange(npad, dtype=jnp.int32) * 97) % n_src)[None, :]
    pad_d = (50000 + (jnp.arange(npad, dtype=jnp.int32) % 48))[None, :]
    src = jnp.concatenate(
        [ei[0].reshape(_NSUB, _ESH),
         jnp.broadcast_to(pad_s, (_NSUB, npad))], axis=1)
    dst = jnp.concatenate(
        [ei[1].reshape(_NSUB, _ESH),
         jnp.broadcast_to(pad_d, (_NSUB, npad))], axis=1)
    return src, dst


def _sc_aggregate(h_sender, h_url, ei_se, ei_ue):
    from jax.experimental.pallas import tpu_sc as plsc

    mesh = plsc.VectorSubcoreMesh(core_axis_name="c", subcore_axis_name="s")
    sse, dse = _pad_edges(ei_se, 10000)
    sue, due = _pad_edges(ei_ue, 50000)

    def locals_for(dst):
        out = []
        for p in range(6):
            plo = p * _PROWS
            inr = (dst >= plo) & (dst < plo + _PROWS)
            out.append(jnp.where(inr, dst - plo,
                                 _PROWS + (dst & 7)).reshape(
                                     _NSUB, _NCHUNK, _CH))
        return out

    zeros = jnp.zeros((_SLAB, 128), jnp.float32)
    agg_kernel = functools.partial(
        pl.kernel, mesh=mesh,
        out_type=jax.ShapeDtypeStruct((_NPADA, 128), jnp.float32),
        scratch_types=[
            pltpu.VMEM((_NCHUNK, _CH), jnp.int32),
            pltpu.VMEM((_NCHUNK, _CH), jnp.int32),
            pltpu.VMEM((_CH, 128), jnp.float32),
            pltpu.VMEM((_CH, 128), jnp.float32),
            pltpu.SemaphoreType.DMA,
            pltpu.SemaphoreType.DMA,
            pltpu.VMEM_SHARED((_SLAB, 128), jnp.float32),
        ])(_sc_agg_body)

    agg_se = agg_kernel(h_sender, sse.reshape(_NSUB, _NCHUNK, _CH),
                        *locals_for(dse), zeros)
    dep = (agg_se[0, 0] * 0.0).astype(jnp.int32)
    agg_ue = agg_kernel(h_url, sue.reshape(_NSUB, _NCHUNK, _CH) + dep,
                        *locals_for(due), zeros)

    ones_e = jnp.ones((_E,), jnp.float32)
    cnt_se = jax.ops.segment_sum(ones_e, ei_se[1], num_segments=50000)
    cnt_ue = jax.ops.segment_sum(ones_e, ei_ue[1], num_segments=50000)
    return agg_se, cnt_se[:, None], agg_ue, cnt_ue[:, None]


def kernel(x_email, x_url, x_sender, edge_index_sender_email,
           edge_index_url_email, edge_index_email_url, W_email, b_email,
           url_base_w, url_spline_w, sender_base_w, sender_spline_w,
           sage_se_wl, sage_se_bl, sage_se_wr, sage_ue_wl, sage_ue_bl,
           sage_ue_wr, sage_eu_wl, sage_eu_bl, sage_eu_wr, bn_gamma,
           bn_beta, cls_base_w, cls_spline_w):
    n_email = x_email.shape[0]
    h = W_email.shape[0]

    h_email = _email_embed(x_email, W_email, b_email)
    h_url = _url_kan(x_url, url_base_w, url_spline_w)
    h_sender = _sender_kan(x_sender, sender_base_w, sender_spline_w)

    agg_se, cnt_se, agg_ue, cnt_ue = _sc_aggregate(
        h_sender, h_url, edge_index_sender_email, edge_index_url_email)

    cse = jnp.clip(cnt_se[:n_email, 0], 1.0, None).reshape(n_email, 1)
    cue = jnp.clip(cnt_ue[:n_email, 0], 1.0, None).reshape(n_email, 1)
    wr_sum = (sage_se_wr + sage_ue_wr).T
    b2 = (0.5 * (sage_se_bl + sage_ue_bl)).reshape(1, h)
    e, stats = _combine_stats(agg_se, cse, agg_ue, cue, h_email,
                              sage_se_wl.T, sage_ue_wl.T, wr_sum, b2)

    inv_n = 1.0 / n_email
    mu = stats[0] * inv_n
    var = stats[1] * inv_n - mu * mu
    scale = (bn_gamma / jnp.sqrt(var + 1e-5)).reshape(1, 2 * h)
    shift = (bn_beta - mu * scale[0]).reshape(1, 2 * h)

    return _classifier(e, h_email, scale, shift, cls_base_w, cls_spline_w)
